# initial kernel scaffold (unmeasured)
import jax
import jax.numpy as jnp
from jax import lax
from jax.experimental import pallas as pl
from jax.experimental.pallas import tpu as pltpu

N_DEV = 4
SQ = 1024
SKV = 1024
HQ = 8
DH = 128
D = HQ * DH
SCALE = 0.08838834764831843


def kernel(x, Wq, K_ext, V_ext, Wo):
    x2 = x.reshape(SQ, D).astype(jnp.bfloat16)
    wq = Wq.astype(jnp.bfloat16)
    wo = Wo.astype(jnp.bfloat16)
    k2 = K_ext.reshape(SKV, D).astype(jnp.bfloat16)
    v2 = V_ext.reshape(SKV, D).astype(jnp.bfloat16)

    def body(x_ref, wq_ref, k_ref, v_ref, wo_ref, out_ref,
             comm_ref, send_sems, recv_sems):
        my = lax.axis_index("i")
        left = lax.rem(my + N_DEV - 1, N_DEV)
        right = lax.rem(my + 1, N_DEV)

        barrier_sem = pltpu.get_barrier_semaphore()
        for nbr in (left, right):
            pl.semaphore_signal(
                barrier_sem, inc=1,
                device_id=(nbr,), device_id_type=pl.DeviceIdType.MESH,
            )
        pl.semaphore_wait(barrier_sem, 2)

        comm_ref[0, :SKV, :] = k_ref[...]
        comm_ref[0, SKV:, :] = v_ref[...]

        for h in range(N_DEV - 1):
            rdma = pltpu.make_async_remote_copy(
                src_ref=comm_ref.at[h],
                dst_ref=comm_ref.at[h + 1],
                send_sem=send_sems.at[h],
                recv_sem=recv_sems.at[h + 1],
                device_id=(right,),
                device_id_type=pl.DeviceIdType.MESH,
            )
            rdma.start()
            rdma.wait()

        q = jnp.dot(x_ref[...], wq_ref[...],
                    preferred_element_type=jnp.float32)
        q = (q * SCALE).astype(jnp.bfloat16)

        r = lax.broadcasted_iota(jnp.int32, (SQ, N_DEV * SKV), 0)
        c = lax.broadcasted_iota(jnp.int32, (SQ, N_DEV * SKV), 1)
        mask = ((r // 64) % 4) == ((c // 64) % 4)

        ctx_parts = []
        for head in range(HQ):
            lo, hi = head * DH, (head + 1) * DH
            qh = q[:, lo:hi]
            kh = jnp.concatenate(
                [comm_ref[s, :SKV, lo:hi] for s in range(N_DEV)], axis=0)
            vh = jnp.concatenate(
                [comm_ref[s, SKV:, lo:hi] for s in range(N_DEV)], axis=0)
            s_h = lax.dot_general(
                qh, kh, (((1,), (1,)), ((), ())),
                preferred_element_type=jnp.float32)
            s_h = jnp.where(mask, s_h, -1e9)
            m = jnp.max(s_h, axis=-1, keepdims=True)
            w = jnp.exp(s_h - m)
            denom = jnp.sum(w, axis=-1, keepdims=True)
            p = (w / denom).astype(jnp.bfloat16)
            ctx_parts.append(lax.dot_general(
                p, vh, (((1,), (0,)), ((), ())),
                preferred_element_type=jnp.float32))
        ctx = jnp.concatenate(ctx_parts, axis=1).astype(jnp.bfloat16)
        out_ref[...] = jnp.dot(ctx, wo_ref[...],
                               preferred_element_type=jnp.float32)

    out = pl.pallas_call(
        body,
        out_shape=jax.ShapeDtypeStruct((SQ, D), jnp.float32),
        in_specs=[pl.BlockSpec(memory_space=pltpu.VMEM)] * 5,
        out_specs=pl.BlockSpec(memory_space=pltpu.VMEM),
        scratch_shapes=[
            pltpu.VMEM((N_DEV, 2 * SKV, D), jnp.bfloat16),
            pltpu.SemaphoreType.DMA((N_DEV,)),
            pltpu.SemaphoreType.DMA((N_DEV,)),
        ],
        compiler_params=pltpu.CompilerParams(collective_id=0),
    )(x2, wq, k2, v2, wo)
    return out.reshape(1, SQ, D)


# baseline (device time: 231812 ns/iter reference)
import jax
import jax.numpy as jnp
from jax import lax
from jax.experimental import pallas as pl
from jax.experimental.pallas import tpu as pltpu

N_DEV = 4
SQ = 1024
SKV = 1024
HQ = 8
DH = 128
D = HQ * DH
SCALE = 0.08838834764831843


def kernel(x, Wq, K_ext, V_ext, Wo):
    x2 = x.reshape(SQ, D).astype(jnp.bfloat16)
    wq = Wq.astype(jnp.bfloat16)
    wo = Wo.astype(jnp.bfloat16)
    k2 = K_ext.reshape(SKV, D).astype(jnp.bfloat16)
    v2 = V_ext.reshape(SKV, D).astype(jnp.bfloat16)

    def body(x_ref, wq_ref, k_ref, v_ref, wo_ref, out_ref,
             comm_ref, send_sems, recv_sems):
        my = lax.axis_index("i")
        left = lax.rem(my + N_DEV - 1, N_DEV)
        right = lax.rem(my + 1, N_DEV)

        barrier_sem = pltpu.get_barrier_semaphore()
        for nbr in (left, right):
            pl.semaphore_signal(
                barrier_sem, inc=1,
                device_id=(nbr,), device_id_type=pl.DeviceIdType.MESH,
            )
        pl.semaphore_wait(barrier_sem, 2)

        comm_ref[0, :SKV, :] = k_ref[...]
        comm_ref[0, SKV:, :] = v_ref[...]

        for h in range(N_DEV - 1):
            rdma = pltpu.make_async_remote_copy(
                src_ref=comm_ref.at[h],
                dst_ref=comm_ref.at[h + 1],
                send_sem=send_sems.at[h],
                recv_sem=recv_sems.at[h + 1],
                device_id=(right,),
                device_id_type=pl.DeviceIdType.MESH,
            )
            rdma.start()
            rdma.wait()

        q = jnp.dot(x_ref[...], wq_ref[...],
                    preferred_element_type=jnp.float32)
        q = (q * SCALE).astype(jnp.bfloat16)

        r = lax.broadcasted_iota(jnp.int32, (SQ, N_DEV * SKV), 0)
        c = lax.broadcasted_iota(jnp.int32, (SQ, N_DEV * SKV), 1)
        mask = ((r // 64) % 4) == ((c // 64) % 4)

        ctx_parts = []
        for head in range(HQ):
            lo, hi = head * DH, (head + 1) * DH
            qh = q[:, lo:hi]
            kh = jnp.concatenate(
                [comm_ref[s, :SKV, lo:hi] for s in range(N_DEV)], axis=0)
            vh = jnp.concatenate(
                [comm_ref[s, SKV:, lo:hi] for s in range(N_DEV)], axis=0)
            s_h = lax.dot_general(
                qh, kh, (((1,), (1,)), ((), ())),
                preferred_element_type=jnp.float32)
            s_h = jnp.where(mask, s_h, -1e9)
            m = jnp.max(s_h, axis=-1, keepdims=True)
            w = jnp.exp(s_h - m)
            denom = jnp.sum(w, axis=-1, keepdims=True)
            p = (w / denom).astype(jnp.bfloat16)
            ctx_parts.append(lax.dot_general(
                p, vh, (((1,), (0,)), ((), ())),
                preferred_element_type=jnp.float32))
        ctx = jnp.concatenate(ctx_parts, axis=1).astype(jnp.bfloat16)
        out_ref[...] = jnp.dot(ctx, wo_ref[...],
                               preferred_element_type=jnp.float32)

    out = pl.pallas_call(
        body,
        out_shape=jax.ShapeDtypeStruct((SQ, D), jnp.float32),
        in_specs=[pl.BlockSpec(memory_space=pltpu.VMEM)] * 5,
        out_specs=pl.BlockSpec(memory_space=pltpu.VMEM),
        scratch_shapes=[
            pltpu.VMEM((N_DEV, 2 * SKV, D), jnp.bfloat16),
            pltpu.SemaphoreType.DMA((N_DEV,)),
            pltpu.SemaphoreType.DMA((N_DEV,)),
        ],
        compiler_params=pltpu.CompilerParams(
            collective_id=0, vmem_limit_bytes=100 * 1024 * 1024),
    )(x2, wq, k2, v2, wo)
    return out.reshape(1, SQ, D)


# device time: 190890 ns/iter; 1.2144x vs baseline; 1.2144x over previous
import jax
import jax.numpy as jnp
from jax import lax
from jax.experimental import pallas as pl
from jax.experimental.pallas import tpu as pltpu

N_DEV = 4
SQ = 1024
SKV = 1024
HQ = 8
DH = 128
D = HQ * DH
SCALE = 0.08838834764831843


def kernel(x, Wq, K_ext, V_ext, Wo):
    x2 = x.reshape(SQ, D).astype(jnp.bfloat16)
    wq = Wq.astype(jnp.bfloat16)
    wo = Wo.astype(jnp.bfloat16)
    k2 = K_ext.reshape(SKV, D).astype(jnp.bfloat16)
    v2 = V_ext.reshape(SKV, D).astype(jnp.bfloat16)

    def body(x_ref, wq_ref, k_ref, v_ref, wo_ref, out_ref,
             comm_ref, ctx_ref, send_sems, recv_sems):
        my = lax.axis_index("i")
        left = lax.rem(my + N_DEV - 1, N_DEV)
        right = lax.rem(my + 1, N_DEV)

        barrier_sem = pltpu.get_barrier_semaphore()
        for nbr in (left, right):
            pl.semaphore_signal(
                barrier_sem, inc=1,
                device_id=(nbr,), device_id_type=pl.DeviceIdType.MESH,
            )
        pl.semaphore_wait(barrier_sem, 2)

        comm_ref[0, :SKV, :] = k_ref[...]
        comm_ref[0, SKV:, :] = v_ref[...]

        for h in range(N_DEV - 1):
            rdma = pltpu.make_async_remote_copy(
                src_ref=comm_ref.at[h],
                dst_ref=comm_ref.at[h + 1],
                send_sem=send_sems.at[h],
                recv_sem=recv_sems.at[h + 1],
                device_id=(right,),
                device_id_type=pl.DeviceIdType.MESH,
            )
            rdma.start()
            rdma.wait()

        q = jnp.dot(x_ref[...], wq_ref[...],
                    preferred_element_type=jnp.float32)
        q = (q * SCALE).astype(jnp.bfloat16)

        for rr in range(4):
            rows = [slice(g * 256 + rr * 64, g * 256 + (rr + 1) * 64)
                    for g in range(4)]
            qr = jnp.concatenate([q[sl, :] for sl in rows], axis=0)
            kr = jnp.concatenate(
                [comm_ref[s, sl, :] for s in range(N_DEV) for sl in rows],
                axis=0)
            vr = jnp.concatenate(
                [comm_ref[s, SKV + sl.start:SKV + sl.stop, :]
                 for s in range(N_DEV) for sl in rows],
                axis=0)
            parts = []
            for head in range(HQ):
                lo, hi = head * DH, (head + 1) * DH
                s_h = lax.dot_general(
                    qr[:, lo:hi], kr[:, lo:hi], (((1,), (1,)), ((), ())),
                    preferred_element_type=jnp.float32)
                m = jnp.max(s_h, axis=-1, keepdims=True)
                w = jnp.exp(s_h - m)
                denom = jnp.sum(w, axis=-1, keepdims=True)
                p = w.astype(jnp.bfloat16)
                c = lax.dot_general(
                    p, vr[:, lo:hi], (((1,), (0,)), ((), ())),
                    preferred_element_type=jnp.float32)
                parts.append(c / denom)
            ctx_r = jnp.concatenate(parts, axis=1).astype(jnp.bfloat16)
            for g, sl in enumerate(rows):
                ctx_ref[sl, :] = ctx_r[g * 64:(g + 1) * 64, :]
        out_ref[...] = jnp.dot(ctx_ref[...], wo_ref[...],
                               preferred_element_type=jnp.float32)

    out = pl.pallas_call(
        body,
        out_shape=jax.ShapeDtypeStruct((SQ, D), jnp.float32),
        in_specs=[pl.BlockSpec(memory_space=pltpu.VMEM)] * 5,
        out_specs=pl.BlockSpec(memory_space=pltpu.VMEM),
        scratch_shapes=[
            pltpu.VMEM((N_DEV, 2 * SKV, D), jnp.bfloat16),
            pltpu.VMEM((SQ, D), jnp.bfloat16),
            pltpu.SemaphoreType.DMA((N_DEV,)),
            pltpu.SemaphoreType.DMA((N_DEV,)),
        ],
        compiler_params=pltpu.CompilerParams(
            collective_id=0, vmem_limit_bytes=100 * 1024 * 1024),
    )(x2, wq, k2, v2, wo)
    return out.reshape(1, SQ, D)


# device time: 121246 ns/iter; 1.9119x vs baseline; 1.5744x over previous
import jax
import jax.numpy as jnp
from jax import lax
from jax.experimental import pallas as pl
from jax.experimental.pallas import tpu as pltpu

N_DEV = 4
SQ = 1024
SKV = 1024
HQ = 8
DH = 128
D = HQ * DH
SCALE = 0.08838834764831843


def kernel(x, Wq, K_ext, V_ext, Wo):
    x2 = x.reshape(SQ, D).astype(jnp.bfloat16)
    wq = Wq.astype(jnp.bfloat16)
    wo = Wo.astype(jnp.bfloat16)
    k2 = K_ext.reshape(SKV, D).astype(jnp.bfloat16)
    v2 = V_ext.reshape(SKV, D).astype(jnp.bfloat16)

    def body(x_ref, wq_ref, k_ref, v_ref, wo_ref, out_ref,
             k_all, v_all, ctx_ref, ksend, krecv, vsend, vrecv):
        my = lax.axis_index("i")
        left = lax.rem(my + N_DEV - 1, N_DEV)
        right = lax.rem(my + 1, N_DEV)

        barrier_sem = pltpu.get_barrier_semaphore()
        for nbr in (left, right):
            pl.semaphore_signal(
                barrier_sem, inc=1,
                device_id=(nbr,), device_id_type=pl.DeviceIdType.MESH,
            )
        pl.semaphore_wait(barrier_sem, 2)

        k_all[0, :, :] = k_ref[...]
        v_all[0, :, :] = v_ref[...]

        def start_hop(h):
            krdma = pltpu.make_async_remote_copy(
                src_ref=k_all.at[h], dst_ref=k_all.at[h + 1],
                send_sem=ksend.at[h], recv_sem=krecv.at[h + 1],
                device_id=(right,), device_id_type=pl.DeviceIdType.MESH,
            )
            vrdma = pltpu.make_async_remote_copy(
                src_ref=v_all.at[h], dst_ref=v_all.at[h + 1],
                send_sem=vsend.at[h], recv_sem=vrecv.at[h + 1],
                device_id=(left,), device_id_type=pl.DeviceIdType.MESH,
            )
            krdma.start()
            vrdma.start()
            return krdma, vrdma

        krdma, vrdma = start_hop(0)

        q = jnp.dot(x_ref[...], wq_ref[...],
                    preferred_element_type=jnp.float32)
        q = (q * SCALE).astype(jnp.bfloat16)

        krdma.wait()
        vrdma.wait()
        for h in range(1, N_DEV - 1):
            krdma, vrdma = start_hop(h)
            krdma.wait()
            vrdma.wait()

        for rr in range(4):
            rows = [slice(g * 256 + rr * 64, g * 256 + (rr + 1) * 64)
                    for g in range(4)]
            qr = jnp.concatenate([q[sl, :] for sl in rows], axis=0)
            kr = jnp.concatenate(
                [k_all[s, sl, :] for s in (0, 1, 2, 3) for sl in rows],
                axis=0)
            vr = jnp.concatenate(
                [v_all[s, sl, :] for s in (0, 3, 2, 1) for sl in rows],
                axis=0)
            parts = []
            for head in range(HQ):
                lo, hi = head * DH, (head + 1) * DH
                s_h = lax.dot_general(
                    qr[:, lo:hi], kr[:, lo:hi], (((1,), (1,)), ((), ())),
                    preferred_element_type=jnp.float32)
                m = jnp.max(s_h, axis=-1, keepdims=True)
                w = jnp.exp(s_h - m)
                denom = jnp.sum(w, axis=-1, keepdims=True)
                p = w.astype(jnp.bfloat16)
                c = lax.dot_general(
                    p, vr[:, lo:hi], (((1,), (0,)), ((), ())),
                    preferred_element_type=jnp.float32)
                parts.append(c / denom)
            ctx_r = jnp.concatenate(parts, axis=1).astype(jnp.bfloat16)
            for g, sl in enumerate(rows):
                ctx_ref[sl, :] = ctx_r[g * 64:(g + 1) * 64, :]
        out_ref[...] = jnp.dot(ctx_ref[...], wo_ref[...],
                               preferred_element_type=jnp.float32)

    out = pl.pallas_call(
        body,
        out_shape=jax.ShapeDtypeStruct((SQ, D), jnp.float32),
        in_specs=[pl.BlockSpec(memory_space=pltpu.VMEM)] * 5,
        out_specs=pl.BlockSpec(memory_space=pltpu.VMEM),
        scratch_shapes=[
            pltpu.VMEM((N_DEV, SKV, D), jnp.bfloat16),
            pltpu.VMEM((N_DEV, SKV, D), jnp.bfloat16),
            pltpu.VMEM((SQ, D), jnp.bfloat16),
            pltpu.SemaphoreType.DMA((N_DEV,)),
            pltpu.SemaphoreType.DMA((N_DEV,)),
            pltpu.SemaphoreType.DMA((N_DEV,)),
            pltpu.SemaphoreType.DMA((N_DEV,)),
        ],
        compiler_params=pltpu.CompilerParams(
            collective_id=0, vmem_limit_bytes=100 * 1024 * 1024),
    )(x2, wq, k2, v2, wo)
    return out.reshape(1, SQ, D)


# device time: 111010 ns/iter; 2.0882x vs baseline; 1.0922x over previous
import jax
import jax.numpy as jnp
from jax import lax
from jax.experimental import pallas as pl
from jax.experimental.pallas import tpu as pltpu

N_DEV = 4
SQ = 1024
SKV = 1024
HQ = 8
DH = 128
D = HQ * DH
SCALE = 0.08838834764831843


def kernel(x, Wq, K_ext, V_ext, Wo):
    x2 = x.reshape(SQ, D).astype(jnp.bfloat16)
    wq = Wq.astype(jnp.bfloat16)
    wo = Wo.astype(jnp.bfloat16)
    k2 = K_ext.reshape(SKV, D).astype(jnp.bfloat16)
    v2 = V_ext.reshape(SKV, D).astype(jnp.bfloat16)

    def body(x_ref, wq_ref, k_ref, v_ref, wo_ref, out_ref,
             k_all, v_all, ctx_ref, ksend, krecv, vsend, vrecv):
        my = lax.axis_index("i")
        left = lax.rem(my + N_DEV - 1, N_DEV)
        right = lax.rem(my + 1, N_DEV)

        barrier_sem = pltpu.get_barrier_semaphore()
        for nbr in (left, right):
            pl.semaphore_signal(
                barrier_sem, inc=1,
                device_id=(nbr,), device_id_type=pl.DeviceIdType.MESH,
            )
        pl.semaphore_wait(barrier_sem, 2)

        k_all[0, :, :] = k_ref[...]
        v_all[0, :, :] = v_ref[...]

        def start_hop(h):
            krdma = pltpu.make_async_remote_copy(
                src_ref=k_all.at[h], dst_ref=k_all.at[h + 1],
                send_sem=ksend.at[h], recv_sem=krecv.at[h + 1],
                device_id=(right,), device_id_type=pl.DeviceIdType.MESH,
            )
            vrdma = pltpu.make_async_remote_copy(
                src_ref=v_all.at[h], dst_ref=v_all.at[h + 1],
                send_sem=vsend.at[h], recv_sem=vrecv.at[h + 1],
                device_id=(left,), device_id_type=pl.DeviceIdType.MESH,
            )
            krdma.start()
            vrdma.start()
            return krdma, vrdma

        def rows_of(rr):
            return [slice(g * 256 + rr * 64, g * 256 + (rr + 1) * 64)
                    for g in range(4)]

        state = {}

        def consume_chunk(qrs, kslot, vslot):
            for rr in range(4):
                rows = rows_of(rr)
                kc = jnp.concatenate(
                    [k_all[kslot, sl, :] for sl in rows], axis=0)
                vc = jnp.concatenate(
                    [v_all[vslot, sl, :] for sl in rows], axis=0)
                for head in range(HQ):
                    lo, hi = head * DH, (head + 1) * DH
                    s_ch = lax.dot_general(
                        qrs[rr][:, lo:hi], kc[:, lo:hi],
                        (((1,), (1,)), ((), ())),
                        preferred_element_type=jnp.float32)
                    m_c = jnp.max(s_ch, axis=-1, keepdims=True)
                    if (rr, head) not in state:
                        w = jnp.exp(s_ch - m_c)
                        l = jnp.sum(w, axis=-1, keepdims=True)
                        acc = lax.dot_general(
                            w.astype(jnp.bfloat16), vc[:, lo:hi],
                            (((1,), (0,)), ((), ())),
                            preferred_element_type=jnp.float32)
                        state[(rr, head)] = (m_c, l, acc)
                    else:
                        m_o, l_o, acc_o = state[(rr, head)]
                        m_n = jnp.maximum(m_o, m_c)
                        corr = jnp.exp(m_o - m_n)
                        w = jnp.exp(s_ch - m_n)
                        l = l_o * corr + jnp.sum(w, axis=-1, keepdims=True)
                        acc = acc_o * corr + lax.dot_general(
                            w.astype(jnp.bfloat16), vc[:, lo:hi],
                            (((1,), (0,)), ((), ())),
                            preferred_element_type=jnp.float32)
                        state[(rr, head)] = (m_n, l, acc)

        krdma, vrdma = start_hop(0)
        q = jnp.dot(x_ref[...], wq_ref[...],
                    preferred_element_type=jnp.float32)
        q = (q * SCALE).astype(jnp.bfloat16)
        qrs = [jnp.concatenate([q[sl, :] for sl in rows_of(rr)], axis=0)
               for rr in range(4)]
        consume_chunk(qrs, 0, 0)
        krdma.wait()
        vrdma.wait()

        krdma, vrdma = start_hop(1)
        krdma.wait()
        vrdma.wait()

        krdma, vrdma = start_hop(2)
        consume_chunk(qrs, 2, 2)
        krdma.wait()
        vrdma.wait()

        consume_chunk(qrs, 1, 3)
        consume_chunk(qrs, 3, 1)

        for rr in range(4):
            ctx_r = jnp.concatenate(
                [state[(rr, head)][2] / state[(rr, head)][1]
                 for head in range(HQ)], axis=1).astype(jnp.bfloat16)
            for g, sl in enumerate(rows_of(rr)):
                ctx_ref[sl, :] = ctx_r[g * 64:(g + 1) * 64, :]
        out_ref[...] = jnp.dot(ctx_ref[...], wo_ref[...],
                               preferred_element_type=jnp.float32)

    out = pl.pallas_call(
        body,
        out_shape=jax.ShapeDtypeStruct((SQ, D), jnp.float32),
        in_specs=[pl.BlockSpec(memory_space=pltpu.VMEM)] * 5,
        out_specs=pl.BlockSpec(memory_space=pltpu.VMEM),
        scratch_shapes=[
            pltpu.VMEM((N_DEV, SKV, D), jnp.bfloat16),
            pltpu.VMEM((N_DEV, SKV, D), jnp.bfloat16),
            pltpu.VMEM((SQ, D), jnp.bfloat16),
            pltpu.SemaphoreType.DMA((N_DEV,)),
            pltpu.SemaphoreType.DMA((N_DEV,)),
            pltpu.SemaphoreType.DMA((N_DEV,)),
            pltpu.SemaphoreType.DMA((N_DEV,)),
        ],
        compiler_params=pltpu.CompilerParams(
            collective_id=0, vmem_limit_bytes=100 * 1024 * 1024),
    )(x2, wq, k2, v2, wo)
    return out.reshape(1, SQ, D)


# device time: 106713 ns/iter; 2.1723x vs baseline; 1.0403x over previous
import jax
import jax.numpy as jnp
from jax import lax
from jax.experimental import pallas as pl
from jax.experimental.pallas import tpu as pltpu

N_DEV = 4
SQ = 1024
SKV = 1024
HQ = 8
DH = 128
D = HQ * DH
SCALE = 0.08838834764831843


def kernel(x, Wq, K_ext, V_ext, Wo):
    x2 = x.reshape(SQ, D).astype(jnp.bfloat16)
    wq = Wq.astype(jnp.bfloat16)
    wo = Wo.astype(jnp.bfloat16)
    k2 = K_ext.reshape(SKV, D).astype(jnp.bfloat16)
    v2 = V_ext.reshape(SKV, D).astype(jnp.bfloat16)

    def body(x_ref, wq_ref, k_ref, v_ref, wo_ref, out_ref,
             kv_a, kv_b, ctx_ref, asend, arecv, bsend, brecv):
        my = lax.axis_index("i")
        left = lax.rem(my + N_DEV - 1, N_DEV)
        right = lax.rem(my + 1, N_DEV)

        barrier_sem = pltpu.get_barrier_semaphore()
        for nbr in (left, right):
            pl.semaphore_signal(
                barrier_sem, inc=1,
                device_id=(nbr,), device_id_type=pl.DeviceIdType.MESH,
            )
        pl.semaphore_wait(barrier_sem, 2)

        kv_a[0, :512, :] = k_ref[:512, :]
        kv_a[0, 512:, :] = v_ref[:512, :]
        kv_b[0, :512, :] = k_ref[512:, :]
        kv_b[0, 512:, :] = v_ref[512:, :]

        def start_hop(h):
            ardma = pltpu.make_async_remote_copy(
                src_ref=kv_a.at[h], dst_ref=kv_a.at[h + 1],
                send_sem=asend.at[h], recv_sem=arecv.at[h + 1],
                device_id=(right,), device_id_type=pl.DeviceIdType.MESH,
            )
            brdma = pltpu.make_async_remote_copy(
                src_ref=kv_b.at[h], dst_ref=kv_b.at[h + 1],
                send_sem=bsend.at[h], recv_sem=brecv.at[h + 1],
                device_id=(left,), device_id_type=pl.DeviceIdType.MESH,
            )
            ardma.start()
            brdma.start()
            return ardma, brdma

        def rows_of(rr):
            return [slice(g * 256 + rr * 64, g * 256 + (rr + 1) * 64)
                    for g in range(4)]

        state = {}

        def consume_slot(qrs, s):
            for rr in range(4):
                if s == 0:
                    kc = jnp.concatenate(
                        [k_ref[sl, :] for sl in rows_of(rr)], axis=0)
                    vc = jnp.concatenate(
                        [v_ref[sl, :] for sl in rows_of(rr)], axis=0)
                else:
                    offs = [slice(g * 256 + rr * 64,
                                  g * 256 + (rr + 1) * 64)
                            for g in range(2)]
                    kc = jnp.concatenate(
                        [kv_a[s, sl, :] for sl in offs]
                        + [kv_b[s, sl, :] for sl in offs], axis=0)
                    vc = jnp.concatenate(
                        [kv_a[s, 512 + sl.start:512 + sl.stop, :]
                         for sl in offs]
                        + [kv_b[s, 512 + sl.start:512 + sl.stop, :]
                           for sl in offs], axis=0)
                for head in range(HQ):
                    lo, hi = head * DH, (head + 1) * DH
                    s_ch = lax.dot_general(
                        qrs[rr][:, lo:hi], kc[:, lo:hi],
                        (((1,), (1,)), ((), ())),
                        preferred_element_type=jnp.float32)
                    m_c = jnp.max(s_ch, axis=-1, keepdims=True)
                    if (rr, head) not in state:
                        w = jnp.exp(s_ch - m_c)
                        l = jnp.sum(w, axis=-1, keepdims=True)
                        acc = lax.dot_general(
                            w.astype(jnp.bfloat16), vc[:, lo:hi],
                            (((1,), (0,)), ((), ())),
                            preferred_element_type=jnp.float32)
                        state[(rr, head)] = (m_c, l, acc)
                    else:
                        m_o, l_o, acc_o = state[(rr, head)]
                        m_n = jnp.maximum(m_o, m_c)
                        corr = jnp.exp(m_o - m_n)
                        w = jnp.exp(s_ch - m_n)
                        l = l_o * corr + jnp.sum(w, axis=-1, keepdims=True)
                        acc = acc_o * corr + lax.dot_general(
                            w.astype(jnp.bfloat16), vc[:, lo:hi],
                            (((1,), (0,)), ((), ())),
                            preferred_element_type=jnp.float32)
                        state[(rr, head)] = (m_n, l, acc)

        ardma, brdma = start_hop(0)
        q = jnp.dot(x_ref[...], wq_ref[...],
                    preferred_element_type=jnp.float32)
        q = (q * SCALE).astype(jnp.bfloat16)
        qrs = [jnp.concatenate([q[sl, :] for sl in rows_of(rr)], axis=0)
               for rr in range(4)]
        consume_slot(qrs, 0)
        ardma.wait()
        brdma.wait()

        ardma, brdma = start_hop(1)
        consume_slot(qrs, 1)
        ardma.wait()
        brdma.wait()

        ardma, brdma = start_hop(2)
        consume_slot(qrs, 2)
        ardma.wait()
        brdma.wait()

        consume_slot(qrs, 3)

        for rr in range(4):
            ctx_r = jnp.concatenate(
                [state[(rr, head)][2] / state[(rr, head)][1]
                 for head in range(HQ)], axis=1).astype(jnp.bfloat16)
            for g, sl in enumerate(rows_of(rr)):
                ctx_ref[sl, :] = ctx_r[g * 64:(g + 1) * 64, :]
        out_ref[...] = jnp.dot(ctx_ref[...], wo_ref[...],
                               preferred_element_type=jnp.float32)

    out = pl.pallas_call(
        body,
        out_shape=jax.ShapeDtypeStruct((SQ, D), jnp.float32),
        in_specs=[pl.BlockSpec(memory_space=pltpu.VMEM)] * 5,
        out_specs=pl.BlockSpec(memory_space=pltpu.VMEM),
        scratch_shapes=[
            pltpu.VMEM((N_DEV, SKV, D), jnp.bfloat16),
            pltpu.VMEM((N_DEV, SKV, D), jnp.bfloat16),
            pltpu.VMEM((SQ, D), jnp.bfloat16),
            pltpu.SemaphoreType.DMA((N_DEV,)),
            pltpu.SemaphoreType.DMA((N_DEV,)),
            pltpu.SemaphoreType.DMA((N_DEV,)),
            pltpu.SemaphoreType.DMA((N_DEV,)),
        ],
        compiler_params=pltpu.CompilerParams(
            collective_id=0, vmem_limit_bytes=100 * 1024 * 1024),
    )(x2, wq, k2, v2, wo)
    return out.reshape(1, SQ, D)


# device time: 106685 ns/iter; 2.1729x vs baseline; 1.0003x over previous
import jax
import jax.numpy as jnp
from jax import lax
from jax.experimental import pallas as pl
from jax.experimental.pallas import tpu as pltpu

N_DEV = 4
SQ = 1024
SKV = 1024
HALF = SKV // 2
HQ = 8
DH = 128
D = HQ * DH
SCALE = 0.08838834764831843


def kernel(x, Wq, K_ext, V_ext, Wo):
    x2 = x.reshape(SQ, D).astype(jnp.bfloat16)
    wq = Wq.astype(jnp.bfloat16)
    wo = Wo.astype(jnp.bfloat16)
    k2 = K_ext.reshape(SKV, D).astype(jnp.bfloat16)
    v2 = V_ext.reshape(SKV, D).astype(jnp.bfloat16)

    def body(x_ref, wq_ref, k_ref, v_ref, wo_ref, out_ref,
             ka, va, kb, vb, sems):
        my = lax.axis_index("i")
        left = lax.rem(my + N_DEV - 1, N_DEV)
        right = lax.rem(my + 1, N_DEV)

        barrier_sem = pltpu.get_barrier_semaphore()
        for nbr in (left, right):
            pl.semaphore_signal(
                barrier_sem, inc=1,
                device_id=(nbr,), device_id_type=pl.DeviceIdType.MESH,
            )
        pl.semaphore_wait(barrier_sem, 2)

        def start_hop(h):
            if h == 0:
                srcs = (k_ref.at[pl.ds(0, HALF)], v_ref.at[pl.ds(0, HALF)],
                        k_ref.at[pl.ds(HALF, HALF)],
                        v_ref.at[pl.ds(HALF, HALF)])
            else:
                srcs = (ka.at[h], va.at[h], kb.at[h], vb.at[h])
            dsts = (ka.at[h + 1], va.at[h + 1], kb.at[h + 1], vb.at[h + 1])
            dirs = (right, right, left, left)
            rdmas = []
            for b in range(4):
                rdma = pltpu.make_async_remote_copy(
                    src_ref=srcs[b], dst_ref=dsts[b],
                    send_sem=sems.at[b, 0, h], recv_sem=sems.at[b, 1, h + 1],
                    device_id=(dirs[b],),
                    device_id_type=pl.DeviceIdType.MESH,
                )
                rdma.start()
                rdmas.append(rdma)
            return rdmas

        def rows_of(rr):
            return [slice(g * 256 + rr * 64, g * 256 + (rr + 1) * 64)
                    for g in range(4)]

        state = {}

        def consume_slot(qrs, s):
            for rr in range(4):
                if s == 0:
                    kc = jnp.concatenate(
                        [k_ref[sl, :] for sl in rows_of(rr)], axis=0)
                    vc = jnp.concatenate(
                        [v_ref[sl, :] for sl in rows_of(rr)], axis=0)
                else:
                    offs = [slice(g * 256 + rr * 64,
                                  g * 256 + (rr + 1) * 64)
                            for g in range(2)]
                    kc = jnp.concatenate(
                        [ka[s, sl, :] for sl in offs]
                        + [kb[s, sl, :] for sl in offs], axis=0)
                    vc = jnp.concatenate(
                        [va[s, sl, :] for sl in offs]
                        + [vb[s, sl, :] for sl in offs], axis=0)
                for head in range(HQ):
                    lo, hi = head * DH, (head + 1) * DH
                    s_ch = lax.dot_general(
                        qrs[rr][:, lo:hi], kc[:, lo:hi],
                        (((1,), (1,)), ((), ())),
                        preferred_element_type=jnp.float32)
                    m_c = jnp.max(s_ch, axis=-1, keepdims=True)
                    if (rr, head) not in state:
                        w = jnp.exp(s_ch - m_c)
                        l = jnp.sum(w, axis=-1, keepdims=True)
                        acc = lax.dot_general(
                            w.astype(jnp.bfloat16), vc[:, lo:hi],
                            (((1,), (0,)), ((), ())),
                            preferred_element_type=jnp.float32)
                        state[(rr, head)] = (m_c, l, acc)
                    else:
                        m_o, l_o, acc_o = state[(rr, head)]
                        m_n = jnp.maximum(m_o, m_c)
                        corr = jnp.exp(m_o - m_n)
                        w = jnp.exp(s_ch - m_n)
                        l = l_o * corr + jnp.sum(w, axis=-1, keepdims=True)
                        acc = acc_o * corr + lax.dot_general(
                            w.astype(jnp.bfloat16), vc[:, lo:hi],
                            (((1,), (0,)), ((), ())),
                            preferred_element_type=jnp.float32)
                        state[(rr, head)] = (m_n, l, acc)

        rdmas = start_hop(0)
        q = jnp.dot(x_ref[...], wq_ref[...],
                    preferred_element_type=jnp.float32)
        q = (q * SCALE).astype(jnp.bfloat16)
        qrs = [jnp.concatenate([q[sl, :] for sl in rows_of(rr)], axis=0)
               for rr in range(4)]
        consume_slot(qrs, 0)
        for r in rdmas:
            r.wait()

        rdmas = start_hop(1)
        consume_slot(qrs, 1)
        for r in rdmas:
            r.wait()

        rdmas = start_hop(2)
        consume_slot(qrs, 2)
        for r in rdmas:
            r.wait()

        consume_slot(qrs, 3)

        fin = []
        for rr in range(4):
            fin.append(jnp.concatenate(
                [state[(rr, head)][2] / state[(rr, head)][1]
                 for head in range(HQ)], axis=1).astype(jnp.bfloat16))
        ctx = jnp.concatenate(
            [fin[rr][g * 64:(g + 1) * 64, :]
             for g in range(4) for rr in range(4)], axis=0)
        out_ref[...] = jnp.dot(ctx, wo_ref[...],
                               preferred_element_type=jnp.float32)

    out = pl.pallas_call(
        body,
        out_shape=jax.ShapeDtypeStruct((SQ, D), jnp.float32),
        in_specs=[pl.BlockSpec(memory_space=pltpu.VMEM)] * 5,
        out_specs=pl.BlockSpec(memory_space=pltpu.VMEM),
        scratch_shapes=[
            pltpu.VMEM((N_DEV, HALF, D), jnp.bfloat16),
            pltpu.VMEM((N_DEV, HALF, D), jnp.bfloat16),
            pltpu.VMEM((N_DEV, HALF, D), jnp.bfloat16),
            pltpu.VMEM((N_DEV, HALF, D), jnp.bfloat16),
            pltpu.SemaphoreType.DMA((4, 2, N_DEV)),
        ],
        compiler_params=pltpu.CompilerParams(
            collective_id=0, vmem_limit_bytes=100 * 1024 * 1024),
    )(x2, wq, k2, v2, wo)
    return out.reshape(1, SQ, D)


# device time: 103219 ns/iter; 2.2458x vs baseline; 1.0336x over previous
import jax
import jax.numpy as jnp
from jax import lax
from jax.experimental import pallas as pl
from jax.experimental.pallas import tpu as pltpu

N_DEV = 4
SQ = 1024
SKV = 1024
HALF = SKV // 2
HQ = 8
DH = 128
D = HQ * DH
SCALE = 0.08838834764831843


def kernel(x, Wq, K_ext, V_ext, Wo):
    x2 = x.reshape(SQ, D).astype(jnp.bfloat16)
    wq = Wq.astype(jnp.bfloat16)
    wo = Wo.astype(jnp.bfloat16)
    k2 = K_ext.reshape(SKV, D).astype(jnp.bfloat16)
    v2 = V_ext.reshape(SKV, D).astype(jnp.bfloat16)

    def body(x_ref, wq_ref, k_ref, v_ref, wo_ref, out_ref,
             ka, va, kb, vb, sems):
        my = lax.axis_index("i")
        left = lax.rem(my + N_DEV - 1, N_DEV)
        right = lax.rem(my + 1, N_DEV)

        barrier_sem = pltpu.get_barrier_semaphore()
        for nbr in (left, right):
            pl.semaphore_signal(
                barrier_sem, inc=1,
                device_id=(nbr,), device_id_type=pl.DeviceIdType.MESH,
            )
        pl.semaphore_wait(barrier_sem, 2)

        def start_one(b, h):
            if h == 0:
                src = (k_ref.at[pl.ds(0, HALF)], v_ref.at[pl.ds(0, HALF)],
                       k_ref.at[pl.ds(HALF, HALF)],
                       v_ref.at[pl.ds(HALF, HALF)])[b]
            else:
                src = (ka, va, kb, vb)[b].at[h]
            rdma = pltpu.make_async_remote_copy(
                src_ref=src, dst_ref=(ka, va, kb, vb)[b].at[h + 1],
                send_sem=sems.at[b, 0, h], recv_sem=sems.at[b, 1, h + 1],
                device_id=((right, right, left, left)[b],),
                device_id_type=pl.DeviceIdType.MESH,
            )
            rdma.start()
            return rdma

        def rows_of(rr):
            return [slice(g * 256 + rr * 64, g * 256 + (rr + 1) * 64)
                    for g in range(4)]

        state = {}

        def consume_slot(qrs, s):
            for rr in range(4):
                if s == 0:
                    kc = jnp.concatenate(
                        [k_ref[sl, :] for sl in rows_of(rr)], axis=0)
                    vc = jnp.concatenate(
                        [v_ref[sl, :] for sl in rows_of(rr)], axis=0)
                else:
                    offs = [slice(g * 256 + rr * 64,
                                  g * 256 + (rr + 1) * 64)
                            for g in range(2)]
                    kc = jnp.concatenate(
                        [ka[s, sl, :] for sl in offs]
                        + [kb[s, sl, :] for sl in offs], axis=0)
                    vc = jnp.concatenate(
                        [va[s, sl, :] for sl in offs]
                        + [vb[s, sl, :] for sl in offs], axis=0)
                for head in range(HQ):
                    lo, hi = head * DH, (head + 1) * DH
                    s_ch = lax.dot_general(
                        qrs[rr][:, lo:hi], kc[:, lo:hi],
                        (((1,), (1,)), ((), ())),
                        preferred_element_type=jnp.float32)
                    m_c = jnp.max(s_ch, axis=-1, keepdims=True)
                    if (rr, head) not in state:
                        w = jnp.exp(s_ch - m_c)
                        l = jnp.sum(w, axis=-1, keepdims=True)
                        acc = lax.dot_general(
                            w.astype(jnp.bfloat16), vc[:, lo:hi],
                            (((1,), (0,)), ((), ())),
                            preferred_element_type=jnp.float32)
                        state[(rr, head)] = (m_c, l, acc)
                    else:
                        m_o, l_o, acc_o = state[(rr, head)]
                        m_n = jnp.maximum(m_o, m_c)
                        corr = jnp.exp(m_o - m_n)
                        w = jnp.exp(s_ch - m_n)
                        l = l_o * corr + jnp.sum(w, axis=-1, keepdims=True)
                        acc = acc_o * corr + lax.dot_general(
                            w.astype(jnp.bfloat16), vc[:, lo:hi],
                            (((1,), (0,)), ((), ())),
                            preferred_element_type=jnp.float32)
                        state[(rr, head)] = (m_n, l, acc)

        r0 = [start_one(b, 0) for b in range(4)]
        q = jnp.dot(x_ref[...], wq_ref[...],
                    preferred_element_type=jnp.float32)
        q = (q * SCALE).astype(jnp.bfloat16)
        qrs = [jnp.concatenate([q[sl, :] for sl in rows_of(rr)], axis=0)
               for rr in range(4)]
        consume_slot(qrs, 0)

        r1 = {}
        for b in (0, 2, 1, 3):
            r0[b].wait()
            r1[b] = start_one(b, 1)
        consume_slot(qrs, 1)

        r2 = {}
        for b in (0, 2, 1, 3):
            r1[b].wait()
            r2[b] = start_one(b, 2)
        consume_slot(qrs, 2)

        for b in (0, 2, 1, 3):
            r2[b].wait()
        consume_slot(qrs, 3)

        fin = []
        for rr in range(4):
            fin.append(jnp.concatenate(
                [state[(rr, head)][2] / state[(rr, head)][1]
                 for head in range(HQ)], axis=1).astype(jnp.bfloat16))
        ctx = jnp.concatenate(
            [fin[rr][g * 64:(g + 1) * 64, :]
             for g in range(4) for rr in range(4)], axis=0)
        out_ref[...] = jnp.dot(ctx, wo_ref[...],
                               preferred_element_type=jnp.float32)

    out = pl.pallas_call(
        body,
        out_shape=jax.ShapeDtypeStruct((SQ, D), jnp.float32),
        in_specs=[pl.BlockSpec(memory_space=pltpu.VMEM)] * 5,
        out_specs=pl.BlockSpec(memory_space=pltpu.VMEM),
        scratch_shapes=[
            pltpu.VMEM((N_DEV, HALF, D), jnp.bfloat16),
            pltpu.VMEM((N_DEV, HALF, D), jnp.bfloat16),
            pltpu.VMEM((N_DEV, HALF, D), jnp.bfloat16),
            pltpu.VMEM((N_DEV, HALF, D), jnp.bfloat16),
            pltpu.SemaphoreType.DMA((4, 2, N_DEV)),
        ],
        compiler_params=pltpu.CompilerParams(
            collective_id=0, vmem_limit_bytes=100 * 1024 * 1024),
    )(x2, wq, k2, v2, wo)
    return out.reshape(1, SQ, D)
